# baseline (speedup 1.0000x reference)
import jax
import jax.numpy as jnp
from jax.experimental import pallas as pl
from jax.experimental.pallas import tpu as pltpu

_B, _N, _DIN, _DHID, _DOUT = 32, 2048, 32, 16, 64
_EPS = 1e-5
_BLK = 256
_NBLK = _N // _BLK
_HB = _DHID * _B

_f32 = jnp.float32
_bf16 = jnp.bfloat16


def _iota_eq(shape, fa, fb):
    a = fa(jax.lax.broadcasted_iota(jnp.int32, shape, 0))
    b = fb(jax.lax.broadcasted_iota(jnp.int32, shape, 1))
    return (a == b).astype(_f32)


def _body(x_ref, adj_ref, w1_ref, w2_ref, b1_ref, b2_ref, scale_ref,
          shift_ref, fcw_ref, fcb_ref, out_ref,
          w1b_ref, w2b_ref, bias1_ref, bias2_ref, xw_ref, s2_ref, acc_ref):
    i = pl.program_id(0)

    @pl.when(i == 0)
    def _build_weights():
        hexp = _iota_eq((_DHID, _HB), lambda r: r, lambda c: c // _B)
        p1 = _iota_eq((_B * _DIN, _DIN), lambda r: r % _DIN, lambda c: c)
        v1 = jnp.dot(p1.astype(_bf16),
                     jnp.dot(w1_ref[...].astype(_bf16), hexp.astype(_bf16),
                             preferred_element_type=_f32).astype(_bf16),
                     preferred_element_type=_f32)
        d1 = _iota_eq((_B * _DIN, _HB), lambda r: r // _DIN, lambda c: c % _B)
        w1b_ref[...] = (v1 * d1).astype(_bf16)
        p2 = _iota_eq((_HB, _DHID), lambda r: r // _B, lambda c: c)
        v2 = jnp.dot(p2.astype(_bf16),
                     jnp.dot(w2_ref[...].astype(_bf16), hexp.astype(_bf16),
                             preferred_element_type=_f32).astype(_bf16),
                     preferred_element_type=_f32)
        d2 = _iota_eq((_HB, _HB), lambda r: r % _B, lambda c: c % _B)
        w2b_ref[...] = (v2 * d2).astype(_bf16)
        bias1_ref[...] = jnp.dot(b1_ref[...], hexp, preferred_element_type=_f32)
        bias2_ref[...] = jnp.dot(b2_ref[...], hexp, preferred_element_type=_f32)

    @pl.when(i < _NBLK)
    def _support1():
        cols = [x_ref[b].astype(_bf16) for b in range(_B)]
        xt_blk = jnp.concatenate(cols, axis=1)
        xw_ref[pl.ds(i * _BLK, _BLK), :] = jnp.dot(
            xt_blk, w1b_ref[...], preferred_element_type=_f32).astype(_bf16)

    @pl.when(jnp.logical_and(i >= _NBLK, i < 2 * _NBLK))
    def _hop1():
        j = i - _NBLK
        h1 = jnp.dot(adj_ref[...].astype(_bf16), xw_ref[...],
                     preferred_element_type=_f32) + bias1_ref[...]
        mean = jnp.mean(h1, axis=1, keepdims=True)
        var = jnp.mean(h1 * h1, axis=1, keepdims=True) - mean * mean
        sc = scale_ref[...] * jax.lax.rsqrt(var + _EPS)
        t = shift_ref[...] - mean * sc
        bnh1 = h1 * sc + t
        s2_ref[pl.ds(j * _BLK, _BLK), :] = jnp.dot(
            bnh1.astype(_bf16), w2b_ref[...],
            preferred_element_type=_f32).astype(_bf16)

    @pl.when(i >= 2 * _NBLK)
    def _hop2_fc():
        h2 = jnp.dot(adj_ref[...].astype(_bf16), s2_ref[...],
                     preferred_element_type=_f32) + bias2_ref[...]
        part = jnp.zeros((_DOUT, _B), dtype=_f32)
        for h in range(_DHID):
            blk = h2[:, _B * h:_B * (h + 1)].astype(_bf16)
            part = part + jnp.dot(fcw_ref[h], blk,
                                  preferred_element_type=_f32)
        prev = jnp.where(i == 2 * _NBLK, jnp.zeros_like(part), acc_ref[...])
        acc = prev + part
        acc_ref[...] = acc

        @pl.when(i == 3 * _NBLK - 1)
        def _():
            out_ref[...] = acc.T + fcb_ref[...]


def _adj_idx(i):
    return (jnp.where(i < _NBLK, 0,
                      jnp.where(i < 2 * _NBLK, i - _NBLK, i - 2 * _NBLK)), 0)


def _row_idx(i):
    return (jnp.clip(i - _NBLK, 0, _NBLK - 1), 0)


def kernel(x, network, W1, b1, W2, b2, gamma, beta, fcW, fcb):
    fcwp = fcW.reshape(_DOUT, _N, _DHID).transpose(2, 0, 1).astype(_bf16)
    return pl.pallas_call(
        _body,
        grid=(3 * _NBLK,),
        in_specs=[
            pl.BlockSpec((_B, _BLK, _DIN),
                         lambda i: (0, jnp.clip(i, 0, _NBLK - 1), 0)),
            pl.BlockSpec((_BLK, _N), _adj_idx),
            pl.BlockSpec((_DIN, _DHID), lambda i: (0, 0)),
            pl.BlockSpec((_DHID, _DHID), lambda i: (0, 0)),
            pl.BlockSpec((1, _DHID), lambda i: (0, 0)),
            pl.BlockSpec((1, _DHID), lambda i: (0, 0)),
            pl.BlockSpec((_BLK, 1), _row_idx),
            pl.BlockSpec((_BLK, 1), _row_idx),
            pl.BlockSpec((_DHID, _DOUT, _BLK),
                         lambda i: (0, 0, jnp.clip(i - 2 * _NBLK, 0,
                                                   _NBLK - 1))),
            pl.BlockSpec((1, _DOUT), lambda i: (0, 0)),
        ],
        out_specs=pl.BlockSpec((_B, _DOUT), lambda i: (0, 0)),
        out_shape=jax.ShapeDtypeStruct((_B, _DOUT), _f32),
        scratch_shapes=[
            pltpu.VMEM((_B * _DIN, _HB), _bf16),
            pltpu.VMEM((_HB, _HB), _bf16),
            pltpu.VMEM((1, _HB), _f32),
            pltpu.VMEM((1, _HB), _f32),
            pltpu.VMEM((_N, _HB), _bf16),
            pltpu.VMEM((_N, _HB), _bf16),
            pltpu.VMEM((_DOUT, _B), _f32),
        ],
        compiler_params=pltpu.CompilerParams(
            dimension_semantics=("arbitrary",)),
    )(x, network, W1, W2, b1[None, :], b2[None, :], gamma[:, None],
      beta[:, None], fcwp, fcb[None, :])


# speedup vs baseline: 1.0551x; 1.0551x over previous
import jax
import jax.numpy as jnp
from jax.experimental import pallas as pl
from jax.experimental.pallas import tpu as pltpu

_B, _N, _DIN, _DHID, _DOUT = 32, 2048, 32, 16, 64
_EPS = 1e-5
_HB = _DHID * _B

_ABLK = 1024
_BBLK = 512
_NB_B = _N // _BBLK
_CBLK = 4096
_NB_C = _N * _DHID // _CBLK

_f32 = jnp.float32
_bf16 = jnp.bfloat16


def _iota_eq(shape, fa, fb):
    a = fa(jax.lax.broadcasted_iota(jnp.int32, shape, 0))
    b = fb(jax.lax.broadcasted_iota(jnp.int32, shape, 1))
    return (a == b).astype(_f32)


def _body_a(x_ref, w1_ref, w2_ref, b1_ref, xw_ref, w2b_ref, bias1_ref):
    hexp = _iota_eq((_DHID, _HB), lambda r: r, lambda c: c // _B)
    p1 = _iota_eq((_B * _DIN, _DIN), lambda r: r % _DIN, lambda c: c)
    v1 = jnp.dot(p1.astype(_bf16),
                 jnp.dot(w1_ref[...].astype(_bf16), hexp.astype(_bf16),
                         preferred_element_type=_f32).astype(_bf16),
                 preferred_element_type=_f32)
    d1 = _iota_eq((_B * _DIN, _HB), lambda r: r // _DIN, lambda c: c % _B)
    w1b = (v1 * d1).astype(_bf16)
    xt_blk = jnp.concatenate([x_ref[b].astype(_bf16) for b in range(_B)],
                             axis=1)
    xw_ref[...] = jnp.dot(xt_blk, w1b,
                          preferred_element_type=_f32).astype(_bf16)
    p2 = _iota_eq((_HB, _DHID), lambda r: r // _B, lambda c: c)
    v2 = jnp.dot(p2.astype(_bf16),
                 jnp.dot(w2_ref[...].astype(_bf16), hexp.astype(_bf16),
                         preferred_element_type=_f32).astype(_bf16),
                 preferred_element_type=_f32)
    d2 = _iota_eq((_HB, _HB), lambda r: r % _B, lambda c: c % _B)
    w2b_ref[...] = (v2 * d2).astype(_bf16)
    bias1_ref[...] = jnp.dot(b1_ref[...], hexp, preferred_element_type=_f32)


def _body_b(adjr_ref, adjc_ref, xw_ref, w2b_ref, scale_ref, shift_ref,
            bias1_ref, h2_ref, acc_ref):
    i = pl.program_id(0)
    h1 = jnp.dot(adjr_ref[...].astype(_bf16), xw_ref[...],
                 preferred_element_type=_f32) + bias1_ref[...]
    mean = jnp.mean(h1, axis=1, keepdims=True)
    var = jnp.mean(h1 * h1, axis=1, keepdims=True) - mean * mean
    sc = scale_ref[...] * jax.lax.rsqrt(var + _EPS)
    t = shift_ref[...] - mean * sc
    bnh1 = h1 * sc + t
    s2j = jnp.dot(bnh1.astype(_bf16), w2b_ref[...],
                  preferred_element_type=_f32).astype(_bf16)
    part = jnp.dot(adjc_ref[...].astype(_bf16), s2j,
                   preferred_element_type=_f32)
    prev = jnp.where(i == 0, jnp.zeros_like(part), acc_ref[...])
    acc = prev + part
    acc_ref[...] = acc

    @pl.when(i == _NB_B - 1)
    def _():
        h2_ref[...] = acc.astype(_bf16)


def _body_c(h2f_ref, fcw_ref, b2_ref, fcb_ref, out_ref, acc_ref):
    i = pl.program_id(0)
    fcw = fcw_ref[...]
    part = jnp.dot(fcw.astype(_bf16), h2f_ref[...],
                   preferred_element_type=_f32)
    sel = _iota_eq((_DHID, _CBLK), lambda r: r, lambda c: c % _DHID)
    b2f = jnp.dot(b2_ref[...], sel, preferred_element_type=_f32)
    cvec = jnp.sum(fcw * b2f, axis=1, keepdims=True)
    prev = jnp.where(i == 0, jnp.zeros_like(part), acc_ref[...])
    acc = prev + part + cvec
    acc_ref[...] = acc
    out_ref[...] = acc.T + fcb_ref[...]


def kernel(x, network, W1, b1, W2, b2, gamma, beta, fcW, fcb):
    xw, w2b, bias1 = pl.pallas_call(
        _body_a,
        grid=(_N // _ABLK,),
        in_specs=[
            pl.BlockSpec((_B, _ABLK, _DIN), lambda i: (0, i, 0)),
            pl.BlockSpec((_DIN, _DHID), lambda i: (0, 0)),
            pl.BlockSpec((_DHID, _DHID), lambda i: (0, 0)),
            pl.BlockSpec((1, _DHID), lambda i: (0, 0)),
        ],
        out_specs=(
            pl.BlockSpec((_ABLK, _HB), lambda i: (i, 0)),
            pl.BlockSpec((_HB, _HB), lambda i: (0, 0)),
            pl.BlockSpec((1, _HB), lambda i: (0, 0)),
        ),
        out_shape=(
            jax.ShapeDtypeStruct((_N, _HB), _bf16),
            jax.ShapeDtypeStruct((_HB, _HB), _bf16),
            jax.ShapeDtypeStruct((1, _HB), _f32),
        ),
        compiler_params=pltpu.CompilerParams(
            dimension_semantics=("arbitrary",)),
    )(x, W1, W2, b1[None, :])

    h2 = pl.pallas_call(
        _body_b,
        grid=(_NB_B,),
        in_specs=[
            pl.BlockSpec((_BBLK, _N), lambda i: (i, 0)),
            pl.BlockSpec((_N, _BBLK), lambda i: (0, i)),
            pl.BlockSpec((_N, _HB), lambda i: (0, 0)),
            pl.BlockSpec((_HB, _HB), lambda i: (0, 0)),
            pl.BlockSpec((_BBLK, 1), lambda i: (i, 0)),
            pl.BlockSpec((_BBLK, 1), lambda i: (i, 0)),
            pl.BlockSpec((1, _HB), lambda i: (0, 0)),
        ],
        out_specs=pl.BlockSpec((_N, _HB), lambda i: (0, 0)),
        out_shape=jax.ShapeDtypeStruct((_N, _HB), _bf16),
        scratch_shapes=[pltpu.VMEM((_N, _HB), _f32)],
        compiler_params=pltpu.CompilerParams(
            dimension_semantics=("arbitrary",)),
    )(network, network, xw, w2b, gamma[:, None], beta[:, None], bias1)

    h2f = h2.reshape(_N * _DHID, _B)

    return pl.pallas_call(
        _body_c,
        grid=(_NB_C,),
        in_specs=[
            pl.BlockSpec((_CBLK, _B), lambda i: (i, 0)),
            pl.BlockSpec((_DOUT, _CBLK), lambda i: (0, i)),
            pl.BlockSpec((1, _DHID), lambda i: (0, 0)),
            pl.BlockSpec((1, _DOUT), lambda i: (0, 0)),
        ],
        out_specs=pl.BlockSpec((_B, _DOUT), lambda i: (0, 0)),
        out_shape=jax.ShapeDtypeStruct((_B, _DOUT), _f32),
        scratch_shapes=[pltpu.VMEM((_DOUT, _B), _f32)],
        compiler_params=pltpu.CompilerParams(
            dimension_semantics=("arbitrary",)),
    )(h2f, fcW, b2[None, :], fcb[None, :])


# speedup vs baseline: 1.4149x; 1.3410x over previous
import jax
import jax.numpy as jnp
from jax.experimental import pallas as pl
from jax.experimental.pallas import tpu as pltpu

_B, _N, _DIN, _DHID, _DOUT = 32, 2048, 32, 16, 64
_EPS = 1e-5
_HB = _DHID * _B

_BBLK = 512
_NB_B = _N // _BBLK
_CBLK = 4096
_NB_C = _N * _DHID // _CBLK

_f32 = jnp.float32
_bf16 = jnp.bfloat16


def _iota_eq(shape, fa, fb):
    a = fa(jax.lax.broadcasted_iota(jnp.int32, shape, 0))
    b = fb(jax.lax.broadcasted_iota(jnp.int32, shape, 1))
    return (a == b).astype(_f32)


def _body_a(xt_ref, w1_ref, w2_ref, b1_ref, xw_ref, w2b_ref, bias1_ref):
    hexp = _iota_eq((_DHID, _HB), lambda r: r, lambda c: c // _B)
    p1 = _iota_eq((_B * _DIN, _DIN), lambda r: r % _DIN, lambda c: c)
    v1 = jnp.dot(p1.astype(_bf16),
                 jnp.dot(w1_ref[...].astype(_bf16), hexp.astype(_bf16),
                         preferred_element_type=_f32).astype(_bf16),
                 preferred_element_type=_f32)
    d1 = _iota_eq((_B * _DIN, _HB), lambda r: r // _DIN, lambda c: c % _B)
    xw_ref[...] = jnp.dot(xt_ref[...], (v1 * d1).astype(_bf16),
                          preferred_element_type=_f32).astype(_bf16)
    p2 = _iota_eq((_HB, _DHID), lambda r: r // _B, lambda c: c)
    v2 = jnp.dot(p2.astype(_bf16),
                 jnp.dot(w2_ref[...].astype(_bf16), hexp.astype(_bf16),
                         preferred_element_type=_f32).astype(_bf16),
                 preferred_element_type=_f32)
    d2 = _iota_eq((_HB, _HB), lambda r: r % _B, lambda c: c % _B)
    w2b_ref[...] = (v2 * d2).astype(_bf16)
    bias1_ref[...] = jnp.dot(b1_ref[...], hexp, preferred_element_type=_f32)


def _body_b(adjr_ref, adjc_ref, xw_ref, w2b_ref, scale_ref, shift_ref,
            bias1_ref, h2_ref, acc_ref):
    i = pl.program_id(0)
    h1 = jnp.dot(adjr_ref[...].astype(_bf16), xw_ref[...],
                 preferred_element_type=_f32) + bias1_ref[...]
    mean = jnp.mean(h1, axis=1, keepdims=True)
    var = jnp.mean(h1 * h1, axis=1, keepdims=True) - mean * mean
    sc = scale_ref[...] * jax.lax.rsqrt(var + _EPS)
    t = shift_ref[...] - mean * sc
    bnh1 = h1 * sc + t
    s2j = jnp.dot(bnh1.astype(_bf16), w2b_ref[...],
                  preferred_element_type=_f32).astype(_bf16)
    part = jnp.dot(adjc_ref[...].astype(_bf16), s2j,
                   preferred_element_type=_f32)
    prev = jnp.where(i == 0, jnp.zeros_like(part), acc_ref[...])
    acc = prev + part
    acc_ref[...] = acc

    @pl.when(i == _NB_B - 1)
    def _():
        h2_ref[...] = acc.astype(_bf16)


def _body_c(h2f_ref, fcw_ref, b2_ref, fcb_ref, out_ref, acc_ref):
    i = pl.program_id(0)
    fcw = fcw_ref[...]
    part = jnp.dot(fcw.astype(_bf16), h2f_ref[...],
                   preferred_element_type=_f32)
    sel = _iota_eq((_DHID, _CBLK), lambda r: r, lambda c: c % _DHID)
    b2f = jnp.dot(b2_ref[...], sel, preferred_element_type=_f32)
    cvec = jnp.sum(fcw * b2f, axis=1, keepdims=True)
    prev = jnp.where(i == 0, jnp.zeros_like(part), acc_ref[...])
    acc = prev + part + cvec
    acc_ref[...] = acc
    out_ref[...] = acc.T + fcb_ref[...]


def kernel(x, network, W1, b1, W2, b2, gamma, beta, fcW, fcb):
    xt = jnp.transpose(x, (1, 0, 2)).reshape(_N, _B * _DIN).astype(_bf16)

    xw, w2b, bias1 = pl.pallas_call(
        _body_a,
        out_shape=(
            jax.ShapeDtypeStruct((_N, _HB), _bf16),
            jax.ShapeDtypeStruct((_HB, _HB), _bf16),
            jax.ShapeDtypeStruct((1, _HB), _f32),
        ),
    )(xt, W1, W2, b1[None, :])

    h2 = pl.pallas_call(
        _body_b,
        grid=(_NB_B,),
        in_specs=[
            pl.BlockSpec((_BBLK, _N), lambda i: (i, 0)),
            pl.BlockSpec((_N, _BBLK), lambda i: (0, i)),
            pl.BlockSpec((_N, _HB), lambda i: (0, 0)),
            pl.BlockSpec((_HB, _HB), lambda i: (0, 0)),
            pl.BlockSpec((_BBLK, 1), lambda i: (i, 0)),
            pl.BlockSpec((_BBLK, 1), lambda i: (i, 0)),
            pl.BlockSpec((1, _HB), lambda i: (0, 0)),
        ],
        out_specs=pl.BlockSpec((_N, _HB), lambda i: (0, 0)),
        out_shape=jax.ShapeDtypeStruct((_N, _HB), _bf16),
        scratch_shapes=[pltpu.VMEM((_N, _HB), _f32)],
        compiler_params=pltpu.CompilerParams(
            dimension_semantics=("arbitrary",)),
    )(network, network, xw, w2b, gamma[:, None], beta[:, None], bias1)

    h2f = h2.reshape(_N * _DHID, _B)

    return pl.pallas_call(
        _body_c,
        grid=(_NB_C,),
        in_specs=[
            pl.BlockSpec((_CBLK, _B), lambda i: (i, 0)),
            pl.BlockSpec((_DOUT, _CBLK), lambda i: (0, i)),
            pl.BlockSpec((1, _DHID), lambda i: (0, 0)),
            pl.BlockSpec((1, _DOUT), lambda i: (0, 0)),
        ],
        out_specs=pl.BlockSpec((_B, _DOUT), lambda i: (0, 0)),
        out_shape=jax.ShapeDtypeStruct((_B, _DOUT), _f32),
        scratch_shapes=[pltpu.VMEM((_DOUT, _B), _f32)],
        compiler_params=pltpu.CompilerParams(
            dimension_semantics=("arbitrary",)),
    )(h2f, fcW, b2[None, :], fcb[None, :])


# speedup vs baseline: 1.4722x; 1.0405x over previous
import jax
import jax.numpy as jnp
from jax.experimental import pallas as pl
from jax.experimental.pallas import tpu as pltpu

_B, _N, _DIN, _DHID, _DOUT = 32, 2048, 32, 16, 64
_EPS = 1e-5
_HB = _DHID * _B

_BBLK = 512
_NB_B = _N // _BBLK
_CBLK = 4096
_NB_C = _N * _DHID // _CBLK

_f32 = jnp.float32
_bf16 = jnp.bfloat16


def _iota_eq(shape, fa, fb):
    a = fa(jax.lax.broadcasted_iota(jnp.int32, shape, 0))
    b = fb(jax.lax.broadcasted_iota(jnp.int32, shape, 1))
    return (a == b).astype(_f32)


def _body_a(xt_ref, w1_ref, w2_ref, b1_ref, xw_ref, w2b_ref, bias1_ref):
    hexp = _iota_eq((_DHID, _HB), lambda r: r, lambda c: c // _B)
    p1 = _iota_eq((_B * _DIN, _DIN), lambda r: r % _DIN, lambda c: c)
    v1 = jnp.dot(p1.astype(_bf16),
                 jnp.dot(w1_ref[...].astype(_bf16), hexp.astype(_bf16),
                         preferred_element_type=_f32).astype(_bf16),
                 preferred_element_type=_f32)
    d1 = _iota_eq((_B * _DIN, _HB), lambda r: r // _DIN, lambda c: c % _B)
    xw_ref[...] = jnp.dot(xt_ref[...], (v1 * d1).astype(_bf16),
                          preferred_element_type=_f32).astype(_bf16)
    p2 = _iota_eq((_HB, _DHID), lambda r: r // _B, lambda c: c)
    v2 = jnp.dot(p2.astype(_bf16),
                 jnp.dot(w2_ref[...].astype(_bf16), hexp.astype(_bf16),
                         preferred_element_type=_f32).astype(_bf16),
                 preferred_element_type=_f32)
    d2 = _iota_eq((_HB, _HB), lambda r: r % _B, lambda c: c % _B)
    w2b_ref[...] = (v2 * d2).astype(_bf16)
    bias1_ref[...] = jnp.dot(b1_ref[...], hexp, preferred_element_type=_f32)


def _body_b(adjr_ref, adjc_ref, xt_ref, w1_ref, w2_ref, b1_ref,
            scale_ref, shift_ref, h2_ref, acc_ref, xw_ref, w2b_ref,
            bias1_ref):
    i = pl.program_id(0)

    @pl.when(i == 0)
    def _prep():
        _body_a(xt_ref, w1_ref, w2_ref, b1_ref, xw_ref, w2b_ref, bias1_ref)

    h1 = jnp.dot(adjr_ref[...].astype(_bf16), xw_ref[...],
                 preferred_element_type=_f32) + bias1_ref[...]
    mean = jnp.mean(h1, axis=1, keepdims=True)
    var = jnp.mean(h1 * h1, axis=1, keepdims=True) - mean * mean
    sc = scale_ref[...] * jax.lax.rsqrt(var + _EPS)
    t = shift_ref[...] - mean * sc
    bnh1 = h1 * sc + t
    s2j = jnp.dot(bnh1.astype(_bf16), w2b_ref[...],
                  preferred_element_type=_f32).astype(_bf16)
    part = jnp.dot(adjc_ref[...].astype(_bf16), s2j,
                   preferred_element_type=_f32)
    prev = jnp.where(i == 0, jnp.zeros_like(part), acc_ref[...])
    acc = prev + part
    acc_ref[...] = acc

    @pl.when(i == _NB_B - 1)
    def _():
        h2_ref[...] = acc.astype(_bf16)


def _body_c(h2f_ref, fcw_ref, b2_ref, fcb_ref, out_ref, acc_ref):
    i = pl.program_id(0)
    fcw = fcw_ref[...]
    part = jnp.dot(fcw.astype(_bf16), h2f_ref[...],
                   preferred_element_type=_f32)
    sel = _iota_eq((_DHID, _CBLK), lambda r: r, lambda c: c % _DHID)
    b2f = jnp.dot(b2_ref[...], sel, preferred_element_type=_f32)
    cvec = jnp.sum(fcw * b2f, axis=1, keepdims=True)
    prev = jnp.where(i == 0, jnp.zeros_like(part), acc_ref[...])
    acc = prev + part + cvec
    acc_ref[...] = acc
    out_ref[...] = acc.T + fcb_ref[...]


def kernel(x, network, W1, b1, W2, b2, gamma, beta, fcW, fcb):
    xt = jnp.transpose(x, (1, 0, 2)).reshape(_N, _B * _DIN).astype(_bf16)

    h2 = pl.pallas_call(
        _body_b,
        grid=(_NB_B,),
        in_specs=[
            pl.BlockSpec((_BBLK, _N), lambda i: (i, 0)),
            pl.BlockSpec((_N, _BBLK), lambda i: (0, i)),
            pl.BlockSpec((_N, _B * _DIN), lambda i: (0, 0)),
            pl.BlockSpec((_DIN, _DHID), lambda i: (0, 0)),
            pl.BlockSpec((_DHID, _DHID), lambda i: (0, 0)),
            pl.BlockSpec((1, _DHID), lambda i: (0, 0)),
            pl.BlockSpec((_BBLK, 1), lambda i: (i, 0)),
            pl.BlockSpec((_BBLK, 1), lambda i: (i, 0)),
        ],
        out_specs=pl.BlockSpec((_N, _HB), lambda i: (0, 0)),
        out_shape=jax.ShapeDtypeStruct((_N, _HB), _bf16),
        scratch_shapes=[
            pltpu.VMEM((_N, _HB), _f32),
            pltpu.VMEM((_N, _HB), _bf16),
            pltpu.VMEM((_HB, _HB), _bf16),
            pltpu.VMEM((1, _HB), _f32),
        ],
        compiler_params=pltpu.CompilerParams(
            dimension_semantics=("arbitrary",)),
    )(network, network, xt, W1, W2, b1[None, :],
      gamma[:, None], beta[:, None])

    h2f = h2.reshape(_N * _DHID, _B)

    return pl.pallas_call(
        _body_c,
        grid=(_NB_C,),
        in_specs=[
            pl.BlockSpec((_CBLK, _B), lambda i: (i, 0)),
            pl.BlockSpec((_DOUT, _CBLK), lambda i: (0, i)),
            pl.BlockSpec((1, _DHID), lambda i: (0, 0)),
            pl.BlockSpec((1, _DOUT), lambda i: (0, 0)),
        ],
        out_specs=pl.BlockSpec((_B, _DOUT), lambda i: (0, 0)),
        out_shape=jax.ShapeDtypeStruct((_B, _DOUT), _f32),
        scratch_shapes=[pltpu.VMEM((_DOUT, _B), _f32)],
        compiler_params=pltpu.CompilerParams(
            dimension_semantics=("arbitrary",)),
    )(h2f, fcW, b2[None, :], fcb[None, :])
